# baseline (speedup 1.0000x reference)
import functools

import jax
import jax.numpy as jnp
from jax import lax
from jax.experimental import pallas as pl
from jax.experimental.pallas import tpu as pltpu
from jax.experimental.pallas import tpu_sc as plsc

N = 10000
E = 320000
D = 128

NC = 2
NS = 16
NW = NC * NS

K = 56
NCH = 179
E_PAD = NW * NCH * K
N_PAD = 10112
ROWS_PER_SUB = N_PAD // NS
NZC = ROWS_PER_SUB // K
ZTAIL = ROWS_PER_SUB - NZC * K
NB = 6
RB = 8
LA = 6
JUNK_ROW = N_PAD - 1


def _agg_body(h_hbm, src_hbm, dst_hbm, zeros_hbm, agg_out,
              src_r, dst_r, rows_v, agg_sh, sem_i, sem_g, sem_s):
    c = lax.axis_index("c")
    s = lax.axis_index("s")
    wid = c * NS + s

    pltpu.sync_copy(zeros_hbm, rows_v.at[0])
    base = s * ROWS_PER_SUB
    for r in range(NZC):
        pltpu.sync_copy(rows_v.at[0], agg_sh.at[pl.ds(base + r * K, K)])
    pltpu.sync_copy(rows_v.at[0].at[pl.ds(0, ZTAIL)],
                    agg_sh.at[pl.ds(base + NZC * K, ZTAIL)])
    plsc.subcore_barrier()

    def idx_start(j, slot):
        pltpu.async_copy(src_hbm.at[wid].at[j], src_r.at[slot], sem_i)
        pltpu.async_copy(dst_hbm.at[wid].at[j], dst_r.at[slot], sem_i)

    def idx_wait(j, slot):
        pltpu.make_async_copy(src_hbm.at[wid].at[j], src_r.at[slot],
                              sem_i).wait()
        pltpu.make_async_copy(dst_hbm.at[wid].at[j], dst_r.at[slot],
                              sem_i).wait()

    def gath_start(islot, bslot):
        pltpu.async_copy(h_hbm.at[src_r.at[islot]], rows_v.at[bslot], sem_g)

    def gath_wait(islot, bslot):
        pltpu.make_async_copy(h_hbm.at[src_r.at[islot]], rows_v.at[bslot],
                              sem_g).wait()

    def scat_start(islot, bslot):
        pltpu.async_copy(rows_v.at[bslot], agg_sh.at[dst_r.at[islot]], sem_s,
                         add=True)

    def scat_wait(islot, bslot):
        pltpu.make_async_copy(rows_v.at[bslot], agg_sh.at[dst_r.at[islot]],
                              sem_s).wait()

    for p in range(LA):
        idx_start(p, p)
    for p in range(NB - 1):
        idx_wait(p, p)
        gath_start(p, p)

    def body(j, carry):
        ij = lax.rem(j, RB)
        bj = lax.rem(j, NB)
        gath_wait(ij, bj)
        scat_start(ij, bj)

        @pl.when(j + LA < NCH)
        def _():
            idx_start(j + LA, lax.rem(j + LA, RB))

        @pl.when(j >= 1)
        def _():
            scat_wait(lax.rem(j - 1, RB), lax.rem(j - 1, NB))

        @pl.when(j + NB - 1 < NCH)
        def _():
            i2 = lax.rem(j + NB - 1, RB)
            idx_wait(j + NB - 1, i2)
            gath_start(i2, lax.rem(j + NB - 1, NB))

        return carry

    lax.fori_loop(0, NCH, body, 0)
    scat_wait(lax.rem(NCH - 1, RB), lax.rem(NCH - 1, NB))
    plsc.subcore_barrier()

    for r in range(NZC):
        lo = base + r * K
        buf = rows_v.at[r % NB]
        pltpu.sync_copy(agg_sh.at[pl.ds(lo, K)], buf)
        pltpu.sync_copy(buf, agg_out.at[c].at[pl.ds(lo, K)])
    tbuf = rows_v.at[NB - 1].at[pl.ds(0, ZTAIL)]
    pltpu.sync_copy(agg_sh.at[pl.ds(base + NZC * K, ZTAIL)], tbuf)
    pltpu.sync_copy(tbuf, agg_out.at[c].at[pl.ds(base + NZC * K, ZTAIL)])


@functools.lru_cache(maxsize=None)
def _make_sc_agg():
    mesh = plsc.VectorSubcoreMesh(core_axis_name="c", subcore_axis_name="s",
                                  num_cores=NC, num_subcores=NS)
    return pl.kernel(
        _agg_body,
        out_type=jax.ShapeDtypeStruct((NC, N_PAD, D), jnp.float32),
        mesh=mesh,
        scratch_types=[
            pltpu.VMEM((RB, K), jnp.int32),
            pltpu.VMEM((RB, K), jnp.int32),
            pltpu.VMEM((NB, K, D), jnp.float32),
            pltpu.VMEM_SHARED((N_PAD, D), jnp.float32),
            pltpu.SemaphoreType.DMA,
            pltpu.SemaphoreType.DMA,
            pltpu.SemaphoreType.DMA,
        ],
    )


def _cnt_body(dst_hbm, ones_hbm, cnt_out, dst_v, ones_v, cnt_sh, sem_s):
    c = lax.axis_index("c")
    s = lax.axis_index("s")
    wid = c * NS + s

    pltpu.sync_copy(dst_hbm.at[wid], dst_v)
    pltpu.sync_copy(ones_hbm.at[pl.ds(0, K)], ones_v)
    base = s * ROWS_PER_SUB
    for r in range(NZC):
        pltpu.sync_copy(ones_v, cnt_sh.at[pl.ds(base + r * K, K)])
    pltpu.sync_copy(ones_v.at[pl.ds(0, ZTAIL)],
                    cnt_sh.at[pl.ds(base + NZC * K, ZTAIL)])
    plsc.subcore_barrier()
    pltpu.sync_copy(ones_hbm.at[pl.ds(K, K)], ones_v)

    def chunk(j, carry):
        pltpu.sync_copy(ones_v, cnt_sh.at[dst_v.at[j]], add=True)
        return carry

    lax.fori_loop(0, NCH, chunk, 0)
    plsc.subcore_barrier()

    for r in range(NZC):
        lo = base + r * K
        pltpu.sync_copy(cnt_sh.at[pl.ds(lo, K)], ones_v)
        pltpu.sync_copy(ones_v, cnt_out.at[c].at[pl.ds(lo, K)])
    tbuf = ones_v.at[pl.ds(0, ZTAIL)]
    pltpu.sync_copy(cnt_sh.at[pl.ds(base + NZC * K, ZTAIL)], tbuf)
    pltpu.sync_copy(tbuf, cnt_out.at[c].at[pl.ds(base + NZC * K, ZTAIL)])
    _ = sem_s


@functools.lru_cache(maxsize=None)
def _make_sc_cnt():
    mesh = plsc.VectorSubcoreMesh(core_axis_name="c", subcore_axis_name="s",
                                  num_cores=NC, num_subcores=NS)
    return pl.kernel(
        _cnt_body,
        out_type=jax.ShapeDtypeStruct((NC, N_PAD, D), jnp.float32),
        mesh=mesh,
        scratch_types=[
            pltpu.VMEM((NCH, K), jnp.int32),
            pltpu.VMEM((K, D), jnp.float32),
            pltpu.VMEM_SHARED((N_PAD, D), jnp.float32),
            pltpu.SemaphoreType.DMA,
        ],
    )


def _dense_body(relu, agg_ref, cnt_ref, h_ref, wl_ref, wr_ref, b_ref, o_ref):
    agg = agg_ref[0] + agg_ref[1]
    cnt = cnt_ref[0, :, 0:1] + cnt_ref[1, :, 0:1]
    mean = agg / jnp.maximum(cnt, 1.0)
    acc = jnp.dot(mean, wl_ref[...], preferred_element_type=jnp.float32)
    acc = acc + jnp.dot(h_ref[...], wr_ref[...],
                        preferred_element_type=jnp.float32)
    acc = acc + b_ref[...]
    o_ref[...] = jnp.maximum(acc, 0.0) if relu else acc


def _tc_dense(agg2, cnt2, h, wl, wr, b, relu):
    B = N_PAD // 8
    return pl.pallas_call(
        functools.partial(_dense_body, relu),
        out_shape=jax.ShapeDtypeStruct((N_PAD, D), jnp.float32),
        grid=(N_PAD // B,),
        in_specs=[
            pl.BlockSpec((NC, B, D), lambda i: (0, i, 0)),
            pl.BlockSpec((NC, B, D), lambda i: (0, i, 0)),
            pl.BlockSpec((B, D), lambda i: (i, 0)),
            pl.BlockSpec((D, D), lambda i: (0, 0)),
            pl.BlockSpec((D, D), lambda i: (0, 0)),
            pl.BlockSpec((1, D), lambda i: (0, 0)),
        ],
        out_specs=pl.BlockSpec((B, D), lambda i: (i, 0)),
    )(agg2, cnt2, h, wl, wr, b)


def kernel(x, edge_index, Wl0, Wr0, b0, Wl1, Wr1, b1, Wl2, Wr2, b2):
    src = edge_index[0]
    dst = edge_index[1]
    pad_e = E_PAD - E
    srcp = jnp.concatenate(
        [src, jnp.zeros((pad_e,), jnp.int32)]).reshape(NW, NCH, K)
    dstp = jnp.concatenate(
        [dst, jnp.full((pad_e,), JUNK_ROW, jnp.int32)]).reshape(NW, NCH, K)
    xp = jnp.pad(x, ((0, N_PAD - N), (0, 0)))
    zeros_blk = jnp.zeros((K, D), jnp.float32)
    zeros_ones_blk = jnp.concatenate(
        [jnp.zeros((K, D), jnp.float32), jnp.ones((K, D), jnp.float32)])

    cnt2 = _make_sc_cnt()(dstp, zeros_ones_blk)
    agg0 = _make_sc_agg()(xp, srcp, dstp, zeros_blk)
    h1 = _tc_dense(agg0, cnt2, xp, Wl0, Wr0, b0.reshape(1, D), relu=True)
    agg1 = _make_sc_agg()(h1, srcp, dstp, zeros_blk)
    h2 = _tc_dense(agg1, cnt2, h1, Wl1, Wr1, b1.reshape(1, D), relu=True)
    agg2 = _make_sc_agg()(h2, srcp, dstp, zeros_blk)
    h3 = _tc_dense(agg2, cnt2, h2, Wl2, Wr2, b2.reshape(1, D), relu=False)
    return h3[:N]


# speedup vs baseline: 1.1896x; 1.1896x over previous
import functools

import jax
import jax.numpy as jnp
from jax import lax
from jax.experimental import pallas as pl
from jax.experimental.pallas import tpu as pltpu
from jax.experimental.pallas import tpu_sc as plsc

N = 10000
E = 320000
D = 128

NC = 2
NS = 16
NW = NC * NS

K = 72
NCH = 139
E_PAD = NW * NCH * K
N_PAD = 10112
ROWS_PER_SUB = N_PAD // NS
NZC = ROWS_PER_SUB // K
ZTAIL = ROWS_PER_SUB - NZC * K
NB = 5
RB = 8
LA = 6
JUNK_ROW = N_PAD - 1


def _agg_body(h_hbm, src_hbm, dst_hbm, zeros_hbm, agg_out,
              src_r, dst_r, rows_v, agg_sh, sem_i, sem_g, sem_s):
    c = lax.axis_index("c")
    s = lax.axis_index("s")
    wid = c * NS + s

    pltpu.sync_copy(zeros_hbm, rows_v.at[0])
    base = s * ROWS_PER_SUB
    for r in range(NZC):
        pltpu.sync_copy(rows_v.at[0], agg_sh.at[pl.ds(base + r * K, K)])
    pltpu.sync_copy(rows_v.at[0].at[pl.ds(0, ZTAIL)],
                    agg_sh.at[pl.ds(base + NZC * K, ZTAIL)])
    plsc.subcore_barrier()

    def idx_start(j, slot):
        pltpu.async_copy(src_hbm.at[wid].at[j], src_r.at[slot], sem_i)
        pltpu.async_copy(dst_hbm.at[wid].at[j], dst_r.at[slot], sem_i)

    def idx_wait(j, slot):
        pltpu.make_async_copy(src_hbm.at[wid].at[j], src_r.at[slot],
                              sem_i).wait()
        pltpu.make_async_copy(dst_hbm.at[wid].at[j], dst_r.at[slot],
                              sem_i).wait()

    def gath_start(islot, bslot):
        pltpu.async_copy(h_hbm.at[src_r.at[islot]], rows_v.at[bslot], sem_g)

    def gath_wait(islot, bslot):
        pltpu.make_async_copy(h_hbm.at[src_r.at[islot]], rows_v.at[bslot],
                              sem_g).wait()

    def scat_start(islot, bslot):
        pltpu.async_copy(rows_v.at[bslot], agg_sh.at[dst_r.at[islot]], sem_s,
                         add=True)

    def scat_wait(islot, bslot):
        pltpu.make_async_copy(rows_v.at[bslot], agg_sh.at[dst_r.at[islot]],
                              sem_s).wait()

    for p in range(LA):
        idx_start(p, p)
    for p in range(NB - 1):
        idx_wait(p, p)
        gath_start(p, p)

    def body(j, carry):
        ij = lax.rem(j, RB)
        bj = lax.rem(j, NB)
        gath_wait(ij, bj)
        scat_start(ij, bj)

        @pl.when(j + LA < NCH)
        def _():
            idx_start(j + LA, lax.rem(j + LA, RB))

        @pl.when(j >= 1)
        def _():
            scat_wait(lax.rem(j - 1, RB), lax.rem(j - 1, NB))

        @pl.when(j + NB - 1 < NCH)
        def _():
            i2 = lax.rem(j + NB - 1, RB)
            idx_wait(j + NB - 1, i2)
            gath_start(i2, lax.rem(j + NB - 1, NB))

        return carry

    lax.fori_loop(0, NCH, body, 0)
    scat_wait(lax.rem(NCH - 1, RB), lax.rem(NCH - 1, NB))
    plsc.subcore_barrier()

    for r in range(NZC):
        lo = base + r * K
        buf = rows_v.at[r % NB]
        pltpu.sync_copy(agg_sh.at[pl.ds(lo, K)], buf)
        pltpu.sync_copy(buf, agg_out.at[c].at[pl.ds(lo, K)])
    tbuf = rows_v.at[NB - 1].at[pl.ds(0, ZTAIL)]
    pltpu.sync_copy(agg_sh.at[pl.ds(base + NZC * K, ZTAIL)], tbuf)
    pltpu.sync_copy(tbuf, agg_out.at[c].at[pl.ds(base + NZC * K, ZTAIL)])


@functools.lru_cache(maxsize=None)
def _make_sc_agg():
    mesh = plsc.VectorSubcoreMesh(core_axis_name="c", subcore_axis_name="s",
                                  num_cores=NC, num_subcores=NS)
    return pl.kernel(
        _agg_body,
        out_type=jax.ShapeDtypeStruct((NC, N_PAD, D), jnp.float32),
        mesh=mesh,
        scratch_types=[
            pltpu.VMEM((RB, K), jnp.int32),
            pltpu.VMEM((RB, K), jnp.int32),
            pltpu.VMEM((NB, K, D), jnp.float32),
            pltpu.VMEM_SHARED((N_PAD, D), jnp.float32),
            pltpu.SemaphoreType.DMA,
            pltpu.SemaphoreType.DMA,
            pltpu.SemaphoreType.DMA,
        ],
    )


def _cnt_body(dst_hbm, ones_hbm, cnt_out, dst_v, ones_v, cnt_sh, sem_s):
    c = lax.axis_index("c")
    s = lax.axis_index("s")
    wid = c * NS + s

    pltpu.sync_copy(dst_hbm.at[wid], dst_v)
    pltpu.sync_copy(ones_hbm.at[pl.ds(0, K)], ones_v)
    base = s * ROWS_PER_SUB
    for r in range(NZC):
        pltpu.sync_copy(ones_v, cnt_sh.at[pl.ds(base + r * K, K)])
    pltpu.sync_copy(ones_v.at[pl.ds(0, ZTAIL)],
                    cnt_sh.at[pl.ds(base + NZC * K, ZTAIL)])
    plsc.subcore_barrier()
    pltpu.sync_copy(ones_hbm.at[pl.ds(K, K)], ones_v)

    def chunk(j, carry):
        pltpu.sync_copy(ones_v, cnt_sh.at[dst_v.at[j]], add=True)
        return carry

    lax.fori_loop(0, NCH, chunk, 0)
    plsc.subcore_barrier()

    for r in range(NZC):
        lo = base + r * K
        pltpu.sync_copy(cnt_sh.at[pl.ds(lo, K)], ones_v)
        pltpu.sync_copy(ones_v, cnt_out.at[c].at[pl.ds(lo, K)])
    tbuf = ones_v.at[pl.ds(0, ZTAIL)]
    pltpu.sync_copy(cnt_sh.at[pl.ds(base + NZC * K, ZTAIL)], tbuf)
    pltpu.sync_copy(tbuf, cnt_out.at[c].at[pl.ds(base + NZC * K, ZTAIL)])
    _ = sem_s


@functools.lru_cache(maxsize=None)
def _make_sc_cnt():
    mesh = plsc.VectorSubcoreMesh(core_axis_name="c", subcore_axis_name="s",
                                  num_cores=NC, num_subcores=NS)
    return pl.kernel(
        _cnt_body,
        out_type=jax.ShapeDtypeStruct((NC, N_PAD, D), jnp.float32),
        mesh=mesh,
        scratch_types=[
            pltpu.VMEM((NCH, K), jnp.int32),
            pltpu.VMEM((K, D), jnp.float32),
            pltpu.VMEM_SHARED((N_PAD, D), jnp.float32),
            pltpu.SemaphoreType.DMA,
        ],
    )


def _dense_body(relu, agg_ref, cnt_ref, h_ref, wl_ref, wr_ref, b_ref, o_ref):
    agg = agg_ref[0] + agg_ref[1]
    cnt = cnt_ref[0, :, 0:1] + cnt_ref[1, :, 0:1]
    mean = agg / jnp.maximum(cnt, 1.0)
    acc = jnp.dot(mean, wl_ref[...], preferred_element_type=jnp.float32)
    acc = acc + jnp.dot(h_ref[...], wr_ref[...],
                        preferred_element_type=jnp.float32)
    acc = acc + b_ref[...]
    o_ref[...] = jnp.maximum(acc, 0.0) if relu else acc


def _tc_dense(agg2, cnt2, h, wl, wr, b, relu):
    B = N_PAD // 8
    return pl.pallas_call(
        functools.partial(_dense_body, relu),
        out_shape=jax.ShapeDtypeStruct((N_PAD, D), jnp.float32),
        grid=(N_PAD // B,),
        in_specs=[
            pl.BlockSpec((NC, B, D), lambda i: (0, i, 0)),
            pl.BlockSpec((NC, B, D), lambda i: (0, i, 0)),
            pl.BlockSpec((B, D), lambda i: (i, 0)),
            pl.BlockSpec((D, D), lambda i: (0, 0)),
            pl.BlockSpec((D, D), lambda i: (0, 0)),
            pl.BlockSpec((1, D), lambda i: (0, 0)),
        ],
        out_specs=pl.BlockSpec((B, D), lambda i: (i, 0)),
    )(agg2, cnt2, h, wl, wr, b)


def kernel(x, edge_index, Wl0, Wr0, b0, Wl1, Wr1, b1, Wl2, Wr2, b2):
    src = edge_index[0]
    dst = edge_index[1]
    pad_e = E_PAD - E
    srcp = jnp.concatenate(
        [src, jnp.zeros((pad_e,), jnp.int32)]).reshape(NW, NCH, K)
    dstp = jnp.concatenate(
        [dst, jnp.full((pad_e,), JUNK_ROW, jnp.int32)]).reshape(NW, NCH, K)
    xp = jnp.pad(x, ((0, N_PAD - N), (0, 0)))
    zeros_blk = jnp.zeros((K, D), jnp.float32)
    zeros_ones_blk = jnp.concatenate(
        [jnp.zeros((K, D), jnp.float32), jnp.ones((K, D), jnp.float32)])

    cnt2 = _make_sc_cnt()(dstp, zeros_ones_blk)
    agg0 = _make_sc_agg()(xp, srcp, dstp, zeros_blk)
    h1 = _tc_dense(agg0, cnt2, xp, Wl0, Wr0, b0.reshape(1, D), relu=True)
    agg1 = _make_sc_agg()(h1, srcp, dstp, zeros_blk)
    h2 = _tc_dense(agg1, cnt2, h1, Wl1, Wr1, b1.reshape(1, D), relu=True)
    agg2 = _make_sc_agg()(h2, srcp, dstp, zeros_blk)
    h3 = _tc_dense(agg2, cnt2, h2, Wl2, Wr2, b2.reshape(1, D), relu=False)
    return h3[:N]


# speedup vs baseline: 1.2905x; 1.0848x over previous
import functools

import jax
import jax.numpy as jnp
from jax import lax
from jax.experimental import pallas as pl
from jax.experimental.pallas import tpu as pltpu
from jax.experimental.pallas import tpu_sc as plsc

N = 10000
E = 320000
D = 128

NC = 2
NS = 16
NW = NC * NS

K = 72
NCH = 139
E_PAD = NW * NCH * K
N_PAD = 10112
ROWS_PER_SUB = N_PAD // NS
NZC = ROWS_PER_SUB // K
ZTAIL = ROWS_PER_SUB - NZC * K
NB = 5
RB = 8
LA = 6
JUNK_ROW = N_PAD - 1
CW = 16


def _agg_body(h_hbm, src_hbm, dst_hbm, zeros_hbm, agg_out,
              src_r, dst_r, rows_v, agg_sh, sem_i, sem_g, sem_s):
    c = lax.axis_index("c")
    s = lax.axis_index("s")
    wid = c * NS + s

    pltpu.sync_copy(zeros_hbm, rows_v.at[0])
    base = s * ROWS_PER_SUB
    for r in range(NZC):
        pltpu.sync_copy(rows_v.at[0], agg_sh.at[pl.ds(base + r * K, K)])
    pltpu.sync_copy(rows_v.at[0].at[pl.ds(0, ZTAIL)],
                    agg_sh.at[pl.ds(base + NZC * K, ZTAIL)])
    plsc.subcore_barrier()

    def idx_start(j, slot):
        pltpu.async_copy(src_hbm.at[wid].at[j], src_r.at[slot], sem_i)
        pltpu.async_copy(dst_hbm.at[wid].at[j], dst_r.at[slot], sem_i)

    def idx_wait(j, slot):
        pltpu.make_async_copy(src_hbm.at[wid].at[j], src_r.at[slot],
                              sem_i).wait()
        pltpu.make_async_copy(dst_hbm.at[wid].at[j], dst_r.at[slot],
                              sem_i).wait()

    def gath_start(islot, bslot):
        pltpu.async_copy(h_hbm.at[src_r.at[islot]], rows_v.at[bslot], sem_g)

    def gath_wait(islot, bslot):
        pltpu.make_async_copy(h_hbm.at[src_r.at[islot]], rows_v.at[bslot],
                              sem_g).wait()

    def scat_start(islot, bslot):
        pltpu.async_copy(rows_v.at[bslot], agg_sh.at[dst_r.at[islot]], sem_s,
                         add=True)

    def scat_wait(islot, bslot):
        pltpu.make_async_copy(rows_v.at[bslot], agg_sh.at[dst_r.at[islot]],
                              sem_s).wait()

    for p in range(LA):
        idx_start(p, p)
    for p in range(NB - 2):
        idx_wait(p, p)
        gath_start(p, p)

    def body(j, carry):
        ij = lax.rem(j, RB)
        bj = lax.rem(j, NB)
        gath_wait(ij, bj)
        scat_start(ij, bj)

        @pl.when(j >= 2)
        def _():
            scat_wait(lax.rem(j - 2, RB), lax.rem(j - 2, NB))

        @pl.when(j + LA < NCH)
        def _():
            idx_start(j + LA, lax.rem(j + LA, RB))

        @pl.when(j + NB - 2 < NCH)
        def _():
            i2 = lax.rem(j + NB - 2, RB)
            idx_wait(j + NB - 2, i2)
            gath_start(i2, lax.rem(j + NB - 2, NB))

        return carry

    lax.fori_loop(0, NCH, body, 0)
    scat_wait(lax.rem(NCH - 2, RB), lax.rem(NCH - 2, NB))
    scat_wait(lax.rem(NCH - 1, RB), lax.rem(NCH - 1, NB))
    plsc.subcore_barrier()

    for r in range(NZC):
        lo = base + r * K
        buf = rows_v.at[r % NB]
        pltpu.sync_copy(agg_sh.at[pl.ds(lo, K)], buf)
        pltpu.sync_copy(buf, agg_out.at[c].at[pl.ds(lo, K)])
    tbuf = rows_v.at[NB - 1].at[pl.ds(0, ZTAIL)]
    pltpu.sync_copy(agg_sh.at[pl.ds(base + NZC * K, ZTAIL)], tbuf)
    pltpu.sync_copy(tbuf, agg_out.at[c].at[pl.ds(base + NZC * K, ZTAIL)])


@functools.lru_cache(maxsize=None)
def _make_sc_agg():
    mesh = plsc.VectorSubcoreMesh(core_axis_name="c", subcore_axis_name="s",
                                  num_cores=NC, num_subcores=NS)
    return pl.kernel(
        _agg_body,
        out_type=jax.ShapeDtypeStruct((NC, N_PAD, D), jnp.float32),
        mesh=mesh,
        scratch_types=[
            pltpu.VMEM((RB, K), jnp.int32),
            pltpu.VMEM((RB, K), jnp.int32),
            pltpu.VMEM((NB, K, D), jnp.float32),
            pltpu.VMEM_SHARED((N_PAD, D), jnp.float32),
            pltpu.SemaphoreType.DMA,
            pltpu.SemaphoreType.DMA,
            pltpu.SemaphoreType.DMA,
        ],
    )


def _cnt_body(dst_hbm, ones_hbm, cnt_out, dst_v, ones_v, cnt_sh, sem_s):
    c = lax.axis_index("c")
    s = lax.axis_index("s")
    wid = c * NS + s

    pltpu.sync_copy(dst_hbm.at[wid], dst_v)
    pltpu.sync_copy(ones_hbm.at[pl.ds(0, K)], ones_v)
    base = s * ROWS_PER_SUB
    for r in range(NZC):
        pltpu.sync_copy(ones_v, cnt_sh.at[pl.ds(base + r * K, K)])
    pltpu.sync_copy(ones_v.at[pl.ds(0, ZTAIL)],
                    cnt_sh.at[pl.ds(base + NZC * K, ZTAIL)])
    plsc.subcore_barrier()
    pltpu.sync_copy(ones_hbm.at[pl.ds(K, K)], ones_v)

    def chunk(j, carry):
        pltpu.sync_copy(ones_v, cnt_sh.at[dst_v.at[j]], add=True)
        return carry

    lax.fori_loop(0, NCH, chunk, 0)
    plsc.subcore_barrier()

    for r in range(NZC):
        lo = base + r * K
        pltpu.sync_copy(cnt_sh.at[pl.ds(lo, K)], ones_v)
        pltpu.sync_copy(ones_v, cnt_out.at[c].at[pl.ds(lo, K)])
    tbuf = ones_v.at[pl.ds(0, ZTAIL)]
    pltpu.sync_copy(cnt_sh.at[pl.ds(base + NZC * K, ZTAIL)], tbuf)
    pltpu.sync_copy(tbuf, cnt_out.at[c].at[pl.ds(base + NZC * K, ZTAIL)])
    _ = sem_s


@functools.lru_cache(maxsize=None)
def _make_sc_cnt():
    mesh = plsc.VectorSubcoreMesh(core_axis_name="c", subcore_axis_name="s",
                                  num_cores=NC, num_subcores=NS)
    return pl.kernel(
        _cnt_body,
        out_type=jax.ShapeDtypeStruct((NC, N_PAD, CW), jnp.float32),
        mesh=mesh,
        scratch_types=[
            pltpu.VMEM((NCH, K), jnp.int32),
            pltpu.VMEM((K, CW), jnp.float32),
            pltpu.VMEM_SHARED((N_PAD, CW), jnp.float32),
            pltpu.SemaphoreType.DMA,
        ],
    )


def _dense_body(relu, agg_ref, cnt_ref, h_ref, wl_ref, wr_ref, b_ref, o_ref):
    agg = agg_ref[0] + agg_ref[1]
    cnt = cnt_ref[0, :, 0:1] + cnt_ref[1, :, 0:1]
    mean = agg / jnp.maximum(cnt, 1.0)
    acc = jnp.dot(mean, wl_ref[...], preferred_element_type=jnp.float32)
    acc = acc + jnp.dot(h_ref[...], wr_ref[...],
                        preferred_element_type=jnp.float32)
    acc = acc + b_ref[...]
    o_ref[...] = jnp.maximum(acc, 0.0) if relu else acc


def _tc_dense(agg2, cnt2, h, wl, wr, b, relu):
    B = N_PAD // 8
    return pl.pallas_call(
        functools.partial(_dense_body, relu),
        out_shape=jax.ShapeDtypeStruct((N_PAD, D), jnp.float32),
        grid=(N_PAD // B,),
        in_specs=[
            pl.BlockSpec((NC, B, D), lambda i: (0, i, 0)),
            pl.BlockSpec((NC, B, CW), lambda i: (0, i, 0)),
            pl.BlockSpec((B, D), lambda i: (i, 0)),
            pl.BlockSpec((D, D), lambda i: (0, 0)),
            pl.BlockSpec((D, D), lambda i: (0, 0)),
            pl.BlockSpec((1, D), lambda i: (0, 0)),
        ],
        out_specs=pl.BlockSpec((B, D), lambda i: (i, 0)),
    )(agg2, cnt2, h, wl, wr, b)


def kernel(x, edge_index, Wl0, Wr0, b0, Wl1, Wr1, b1, Wl2, Wr2, b2):
    src = edge_index[0]
    dst = edge_index[1]
    pad_e = E_PAD - E
    srcp = jnp.concatenate(
        [src, jnp.zeros((pad_e,), jnp.int32)]).reshape(NW, NCH, K)
    dstp = jnp.concatenate(
        [dst, jnp.full((pad_e,), JUNK_ROW, jnp.int32)]).reshape(NW, NCH, K)
    xp = jnp.pad(x, ((0, N_PAD - N), (0, 0)))
    zeros_blk = jnp.zeros((K, D), jnp.float32)
    zeros_ones_blk = jnp.concatenate(
        [jnp.zeros((K, CW), jnp.float32), jnp.ones((K, CW), jnp.float32)])

    cnt2 = _make_sc_cnt()(dstp, zeros_ones_blk)
    agg0 = _make_sc_agg()(xp, srcp, dstp, zeros_blk)
    h1 = _tc_dense(agg0, cnt2, xp, Wl0, Wr0, b0.reshape(1, D), relu=True)
    agg1 = _make_sc_agg()(h1, srcp, dstp, zeros_blk)
    h2 = _tc_dense(agg1, cnt2, h1, Wl1, Wr1, b1.reshape(1, D), relu=True)
    agg2 = _make_sc_agg()(h2, srcp, dstp, zeros_blk)
    h3 = _tc_dense(agg2, cnt2, h2, Wl2, Wr2, b2.reshape(1, D), relu=False)
    return h3[:N]


# speedup vs baseline: 1.3113x; 1.0161x over previous
import functools

import jax
import jax.numpy as jnp
from jax import lax
from jax.experimental import pallas as pl
from jax.experimental.pallas import tpu as pltpu
from jax.experimental.pallas import tpu_sc as plsc

N = 10000
E = 320000
D = 128

NC = 2
NS = 16
NW = NC * NS

K = 72
NCH = 139
E_PAD = NW * NCH * K
N_PAD = 10112
ROWS_PER_SUB = N_PAD // NS
NZC = ROWS_PER_SUB // K
ZTAIL = ROWS_PER_SUB - NZC * K
NB = 5
RB = 8
LA = 6
JUNK_ROW = N_PAD - 1
CW = 16


def _agg_body(h_hbm, src_hbm, dst_hbm, zeros_hbm, agg_out,
              src_r, dst_r, rows_v, agg_sh, sem_i, sem_g, sem_s):
    c = lax.axis_index("c")
    s = lax.axis_index("s")
    wid = c * NS + s

    pltpu.sync_copy(zeros_hbm, rows_v.at[0])
    base = s * ROWS_PER_SUB
    for r in range(NZC):
        pltpu.sync_copy(rows_v.at[0], agg_sh.at[pl.ds(base + r * K, K)])
    pltpu.sync_copy(rows_v.at[0].at[pl.ds(0, ZTAIL)],
                    agg_sh.at[pl.ds(base + NZC * K, ZTAIL)])
    plsc.subcore_barrier()

    def idx_start(j, slot):
        pltpu.async_copy(src_hbm.at[wid].at[j], src_r.at[slot], sem_i)
        pltpu.async_copy(dst_hbm.at[wid].at[j], dst_r.at[slot], sem_i)

    def idx_wait(j, slot):
        pltpu.make_async_copy(src_hbm.at[wid].at[j], src_r.at[slot],
                              sem_i).wait()
        pltpu.make_async_copy(dst_hbm.at[wid].at[j], dst_r.at[slot],
                              sem_i).wait()

    def gath_start(islot, bslot):
        pltpu.async_copy(h_hbm.at[src_r.at[islot]], rows_v.at[bslot], sem_g)

    def gath_wait(islot, bslot):
        pltpu.make_async_copy(h_hbm.at[src_r.at[islot]], rows_v.at[bslot],
                              sem_g).wait()

    def scat_start(islot, bslot):
        pltpu.async_copy(rows_v.at[bslot], agg_sh.at[dst_r.at[islot]], sem_s,
                         add=True)

    def scat_wait(islot, bslot):
        pltpu.make_async_copy(rows_v.at[bslot], agg_sh.at[dst_r.at[islot]],
                              sem_s).wait()

    for p in range(LA):
        idx_start(p, p)
    for p in range(NB - 1):
        idx_wait(p, p)
        gath_start(p, p)

    def body(j, carry):
        ij = lax.rem(j, RB)
        bj = lax.rem(j, NB)
        gath_wait(ij, bj)
        scat_start(ij, bj)

        @pl.when(j + LA < NCH)
        def _():
            idx_start(j + LA, lax.rem(j + LA, RB))

        @pl.when(j >= 1)
        def _():
            scat_wait(lax.rem(j - 1, RB), lax.rem(j - 1, NB))

        @pl.when(j + NB - 1 < NCH)
        def _():
            i2 = lax.rem(j + NB - 1, RB)
            idx_wait(j + NB - 1, i2)
            gath_start(i2, lax.rem(j + NB - 1, NB))

        return carry

    lax.fori_loop(0, NCH, body, 0)
    scat_wait(lax.rem(NCH - 1, RB), lax.rem(NCH - 1, NB))
    plsc.subcore_barrier()

    pltpu.sync_copy(agg_sh.at[pl.ds(base, ROWS_PER_SUB)],
                    agg_out.at[c].at[pl.ds(base, ROWS_PER_SUB)])


@functools.lru_cache(maxsize=None)
def _make_sc_agg():
    mesh = plsc.VectorSubcoreMesh(core_axis_name="c", subcore_axis_name="s",
                                  num_cores=NC, num_subcores=NS)
    return pl.kernel(
        _agg_body,
        out_type=jax.ShapeDtypeStruct((NC, N_PAD, D), jnp.float32),
        mesh=mesh,
        scratch_types=[
            pltpu.VMEM((RB, K), jnp.int32),
            pltpu.VMEM((RB, K), jnp.int32),
            pltpu.VMEM((NB, K, D), jnp.float32),
            pltpu.VMEM_SHARED((N_PAD, D), jnp.float32),
            pltpu.SemaphoreType.DMA,
            pltpu.SemaphoreType.DMA,
            pltpu.SemaphoreType.DMA,
        ],
    )


def _cnt_body(dst_hbm, ones_hbm, cnt_out, dst_v, ones_v, cnt_sh, sem_s):
    c = lax.axis_index("c")
    s = lax.axis_index("s")
    wid = c * NS + s

    pltpu.sync_copy(dst_hbm.at[wid], dst_v)
    pltpu.sync_copy(ones_hbm.at[pl.ds(0, K)], ones_v)
    base = s * ROWS_PER_SUB
    for r in range(NZC):
        pltpu.sync_copy(ones_v, cnt_sh.at[pl.ds(base + r * K, K)])
    pltpu.sync_copy(ones_v.at[pl.ds(0, ZTAIL)],
                    cnt_sh.at[pl.ds(base + NZC * K, ZTAIL)])
    plsc.subcore_barrier()
    pltpu.sync_copy(ones_hbm.at[pl.ds(K, K)], ones_v)

    def chunk(j, carry):
        pltpu.sync_copy(ones_v, cnt_sh.at[dst_v.at[j]], add=True)
        return carry

    lax.fori_loop(0, NCH, chunk, 0)
    plsc.subcore_barrier()

    pltpu.sync_copy(cnt_sh.at[pl.ds(base, ROWS_PER_SUB)],
                    cnt_out.at[c].at[pl.ds(base, ROWS_PER_SUB)])
    _ = sem_s


@functools.lru_cache(maxsize=None)
def _make_sc_cnt():
    mesh = plsc.VectorSubcoreMesh(core_axis_name="c", subcore_axis_name="s",
                                  num_cores=NC, num_subcores=NS)
    return pl.kernel(
        _cnt_body,
        out_type=jax.ShapeDtypeStruct((NC, N_PAD, CW), jnp.float32),
        mesh=mesh,
        scratch_types=[
            pltpu.VMEM((NCH, K), jnp.int32),
            pltpu.VMEM((K, CW), jnp.float32),
            pltpu.VMEM_SHARED((N_PAD, CW), jnp.float32),
            pltpu.SemaphoreType.DMA,
        ],
    )


def _dense_body(relu, agg_ref, cnt_ref, h_ref, wl_ref, wr_ref, b_ref, o_ref):
    agg = agg_ref[0] + agg_ref[1]
    cnt = cnt_ref[0, :, 0:1] + cnt_ref[1, :, 0:1]
    mean = agg / jnp.maximum(cnt, 1.0)
    acc = jnp.dot(mean, wl_ref[...], preferred_element_type=jnp.float32)
    acc = acc + jnp.dot(h_ref[...], wr_ref[...],
                        preferred_element_type=jnp.float32)
    acc = acc + b_ref[...]
    o_ref[...] = jnp.maximum(acc, 0.0) if relu else acc


def _tc_dense(agg2, cnt2, h, wl, wr, b, relu):
    B = N_PAD // 8
    return pl.pallas_call(
        functools.partial(_dense_body, relu),
        out_shape=jax.ShapeDtypeStruct((N_PAD, D), jnp.float32),
        grid=(N_PAD // B,),
        in_specs=[
            pl.BlockSpec((NC, B, D), lambda i: (0, i, 0)),
            pl.BlockSpec((NC, B, CW), lambda i: (0, i, 0)),
            pl.BlockSpec((B, D), lambda i: (i, 0)),
            pl.BlockSpec((D, D), lambda i: (0, 0)),
            pl.BlockSpec((D, D), lambda i: (0, 0)),
            pl.BlockSpec((1, D), lambda i: (0, 0)),
        ],
        out_specs=pl.BlockSpec((B, D), lambda i: (i, 0)),
    )(agg2, cnt2, h, wl, wr, b)


def kernel(x, edge_index, Wl0, Wr0, b0, Wl1, Wr1, b1, Wl2, Wr2, b2):
    src = edge_index[0]
    dst = edge_index[1]
    pad_e = E_PAD - E
    srcp = jnp.concatenate(
        [src, jnp.zeros((pad_e,), jnp.int32)]).reshape(NW, NCH, K)
    dstp = jnp.concatenate(
        [dst, jnp.full((pad_e,), JUNK_ROW, jnp.int32)]).reshape(NW, NCH, K)
    xp = jnp.pad(x, ((0, N_PAD - N), (0, 0)))
    zeros_blk = jnp.zeros((K, D), jnp.float32)
    zeros_ones_blk = jnp.concatenate(
        [jnp.zeros((K, CW), jnp.float32), jnp.ones((K, CW), jnp.float32)])

    cnt2 = _make_sc_cnt()(dstp, zeros_ones_blk)
    agg0 = _make_sc_agg()(xp, srcp, dstp, zeros_blk)
    h1 = _tc_dense(agg0, cnt2, xp, Wl0, Wr0, b0.reshape(1, D), relu=True)
    agg1 = _make_sc_agg()(h1, srcp, dstp, zeros_blk)
    h2 = _tc_dense(agg1, cnt2, h1, Wl1, Wr1, b1.reshape(1, D), relu=True)
    agg2 = _make_sc_agg()(h2, srcp, dstp, zeros_blk)
    h3 = _tc_dense(agg2, cnt2, h2, Wl2, Wr2, b2.reshape(1, D), relu=False)
    return h3[:N]
